# baseline (speedup 1.0000x reference)
import jax
import jax.numpy as jnp
from jax import lax
from jax.experimental import pallas as pl
from jax.experimental.pallas import tpu as pltpu
from jax.experimental.pallas import tpu_sc as plsc

_K = 8192
_D = 64
_TILE = 512


def _argmin_tile_kernel(f_ref, e_ref, b_ref, i_ref):
    f = f_ref[...]
    e = e_ref[...]
    a = jnp.sum(f ** 2, axis=1, keepdims=True)
    b = b_ref[...]
    c = lax.dot_general(
        f, e, (((1,), (1,)), ((), ())),
        preferred_element_type=jnp.float32)
    d = (a + b) - 2.0 * c
    m = jnp.min(d, axis=1, keepdims=True)
    col = lax.broadcasted_iota(jnp.int32, (_TILE, _K), 1)
    idx = jnp.min(jnp.where(d == m, col, _K), axis=1)
    i_ref[...] = idx[:, None]


def _onehot_tile_kernel(i_ref, enc_ref):
    idx = i_ref[...]
    col = lax.broadcasted_iota(jnp.int32, (_TILE, _K), 1)
    enc_ref[...] = (col == idx).astype(jnp.float32)


def _tc_argmin(flat, embedding, b):
    n = flat.shape[0]
    return pl.pallas_call(
        _argmin_tile_kernel,
        grid=(n // _TILE,),
        in_specs=[
            pl.BlockSpec((_TILE, _D), lambda i: (i, 0)),
            pl.BlockSpec((_K, _D), lambda i: (0, 0)),
            pl.BlockSpec((1, _K), lambda i: (0, 0)),
        ],
        out_specs=pl.BlockSpec((_TILE, 1), lambda i: (i, 0)),
        out_shape=jax.ShapeDtypeStruct((n, 1), jnp.int32),
        compiler_params=pltpu.CompilerParams(
            dimension_semantics=("parallel",)),
    )(flat, embedding, b)


def _tc_onehot(idx):
    n = idx.shape[0]
    return pl.pallas_call(
        _onehot_tile_kernel,
        grid=(n // _TILE,),
        in_specs=[pl.BlockSpec((_TILE, 1), lambda i: (i, 0))],
        out_specs=pl.BlockSpec((_TILE, _K), lambda i: (i, 0)),
        out_shape=jax.ShapeDtypeStruct((n, _K), jnp.float32),
        compiler_params=pltpu.CompilerParams(
            dimension_semantics=("parallel",)),
    )(idx)


def _sc_gather(table, idx, n):
    info = plsc.get_sparse_core_info()
    nw = info.num_cores * info.num_subcores
    b_per_w = n // nw

    def body(table_hbm, idx_hbm, out_hbm, idx_v, rows_v, sem):
        wid = lax.axis_index("s") * info.num_cores + lax.axis_index("c")
        base = wid * b_per_w
        pltpu.sync_copy(idx_hbm.at[pl.ds(base, b_per_w)], idx_v)
        pltpu.async_copy(table_hbm.at[idx_v], rows_v, sem).wait()
        pltpu.sync_copy(rows_v, out_hbm.at[pl.ds(base, b_per_w)])

    return pl.kernel(
        body,
        mesh=plsc.VectorSubcoreMesh(core_axis_name="c", subcore_axis_name="s"),
        out_type=jax.ShapeDtypeStruct((n, _D), jnp.float32),
        scratch_types=[
            pltpu.VMEM((b_per_w,), jnp.int32),
            pltpu.VMEM((b_per_w, _D), jnp.float32),
            pltpu.SemaphoreType.DMA,
        ],
        compiler_params=pltpu.CompilerParams(use_tc_tiling_on_sc=False),
    )(table, idx)


def kernel(features, embedding):
    input_shape = features.shape
    flat = features.reshape(-1, _D)
    n = flat.shape[0]
    b = jnp.sum(embedding ** 2, axis=1)[None, :]

    idx = _tc_argmin(flat, embedding, b)
    q = _sc_gather(embedding, idx.reshape(-1), n)
    enc = _tc_onehot(idx)
    return q.reshape(input_shape), idx, enc


# speedup vs baseline: 1.3014x; 1.3014x over previous
import jax
import jax.numpy as jnp
from jax import lax
from jax.experimental import pallas as pl
from jax.experimental.pallas import tpu as pltpu
from jax.experimental.pallas import tpu_sc as plsc

_K = 8192
_D = 64
_TILE = 512


def _vq_tile_kernel(f_ref, e_ref, b_ref, i_ref, enc_ref):
    f = f_ref[...]
    e = e_ref[...]
    a = jnp.sum(f ** 2, axis=1, keepdims=True)
    b = b_ref[...]
    c = lax.dot_general(
        f, e, (((1,), (1,)), ((), ())),
        preferred_element_type=jnp.float32)
    d = (a + b) - 2.0 * c
    m = jnp.min(d, axis=1, keepdims=True)
    col = lax.broadcasted_iota(jnp.int32, (_TILE, _K), 1)
    idx = jnp.min(jnp.where(d == m, col, _K), axis=1)
    enc_ref[...] = (col == idx[:, None]).astype(jnp.float32)
    i_ref[...] = idx[:, None]


def _tc_argmin_onehot(flat, embedding, b):
    n = flat.shape[0]
    return pl.pallas_call(
        _vq_tile_kernel,
        grid=(n // _TILE,),
        in_specs=[
            pl.BlockSpec((_TILE, _D), lambda i: (i, 0)),
            pl.BlockSpec((_K, _D), lambda i: (0, 0)),
            pl.BlockSpec((1, _K), lambda i: (0, 0)),
        ],
        out_specs=[
            pl.BlockSpec((_TILE, 1), lambda i: (i, 0)),
            pl.BlockSpec((_TILE, _K), lambda i: (i, 0)),
        ],
        out_shape=[
            jax.ShapeDtypeStruct((n, 1), jnp.int32),
            jax.ShapeDtypeStruct((n, _K), jnp.float32),
        ],
        compiler_params=pltpu.CompilerParams(
            dimension_semantics=("parallel",)),
    )(flat, embedding, b)


def _sc_gather(table, idx, n):
    info = plsc.get_sparse_core_info()
    nw = info.num_cores * info.num_subcores
    b_per_w = n // nw

    def body(table_hbm, idx_hbm, out_hbm, idx_v, rows_v, sem):
        wid = lax.axis_index("s") * info.num_cores + lax.axis_index("c")
        base = wid * b_per_w
        pltpu.sync_copy(idx_hbm.at[pl.ds(base, b_per_w)], idx_v)
        pltpu.async_copy(table_hbm.at[idx_v], rows_v, sem).wait()
        pltpu.sync_copy(rows_v, out_hbm.at[pl.ds(base, b_per_w)])

    return pl.kernel(
        body,
        mesh=plsc.VectorSubcoreMesh(core_axis_name="c", subcore_axis_name="s"),
        out_type=jax.ShapeDtypeStruct((n, _D), jnp.float32),
        scratch_types=[
            pltpu.VMEM((b_per_w,), jnp.int32),
            pltpu.VMEM((b_per_w, _D), jnp.float32),
            pltpu.SemaphoreType.DMA,
        ],
        compiler_params=pltpu.CompilerParams(use_tc_tiling_on_sc=False),
    )(table, idx)


def kernel(features, embedding):
    input_shape = features.shape
    flat = features.reshape(-1, _D)
    n = flat.shape[0]
    b = jnp.sum(embedding ** 2, axis=1)[None, :]

    idx, enc = _tc_argmin_onehot(flat, embedding, b)
    q = _sc_gather(embedding, idx.reshape(-1), n)
    return q.reshape(input_shape), idx, enc
